# baseline (speedup 1.0000x reference)
import functools

import jax
import jax.numpy as jnp
from jax import lax
from jax.experimental import pallas as pl
from jax.experimental.pallas import tpu as pltpu
from jax.experimental.pallas import tpu_sc as plsc

_NC = 2
_NS = 16
_NW = _NC * _NS
_L = 16
_D = 64
_CH = 128
_BS = 2048


def _bf16_hi(x):
    u = lax.bitcast_convert_type(x, jnp.uint32)
    r = u + jnp.uint32(0x7FFF) + ((u >> jnp.uint32(16)) & jnp.uint32(1))
    return r & jnp.uint32(0xFFFF0000)


def _prep_body(uepta, uepta2, uenta, uenta2, iepta, iepta2, ienta, ienta2,
               distt, ue2, ie2, d0, d1, d2, d3, d4):
    def pack(lo_t, hi_t):
        word = (_bf16_hi(lo_t.T) >> jnp.uint32(16)) | _bf16_hi(hi_t.T)
        return lax.bitcast_convert_type(word, jnp.int32)

    ue2[:, 0:_D] = pack(uepta[...], uenta[...])
    ue2[:, _D:2 * _D] = pack(uepta2[...], uenta2[...])
    ie2[:, 0:_D] = pack(iepta[...], ienta[...])
    ie2[:, _D:2 * _D] = pack(iepta2[...], ienta2[...])
    dd = distt[...]
    d0[...] = dd[0]
    d1[...] = dd[1]
    d2[...] = dd[2]
    d3[...] = dd[3]
    d4[...] = dd[4]


def _sc_body(users, items, ue2, ie2, rp_tab, ug_tab, ud_tab, up_tab, un_tab,
             ibp_tab, ibn_tab, d0_tab, d1_tab, d2_tab, d3_tab, d4_tab,
             pos_out, neg_out, rpo_out, gam_out, dlt_out, upg_out, ung_out,
             ibpg_out, ibng_out, w0_out, w1_out, w2_out, w3_out, w4_out,
             u_idx, i_idx, u_row, i_row, rp_v, g_v, d_v, up_v, un_v,
             ibp_v, ibn_v, dv0, dv1, dv2, dv3, dv4, pos_v, neg_v,
             ue_b, ie_b, sem_small, sem_a, sem_b, nu2, ni2):
    bpw = pos_v.shape[0]
    nch = bpw // _CH
    wid = lax.axis_index("s") * _NC + lax.axis_index("c")
    base = wid * bpw

    pltpu.sync_copy(users.at[pl.ds(base, bpw)], u_idx)
    pltpu.sync_copy(items.at[pl.ds(base, bpw)], i_idx)

    small = [
        pltpu.async_copy(rp_tab.at[u_idx], rp_v, sem_small),
        pltpu.async_copy(ug_tab.at[u_idx], g_v, sem_small),
        pltpu.async_copy(ud_tab.at[u_idx], d_v, sem_small),
        pltpu.async_copy(up_tab.at[u_idx], up_v, sem_small),
        pltpu.async_copy(un_tab.at[u_idx], un_v, sem_small),
        pltpu.async_copy(ibp_tab.at[i_idx], ibp_v, sem_small),
        pltpu.async_copy(ibn_tab.at[i_idx], ibn_v, sem_small),
        pltpu.async_copy(d0_tab.at[i_idx], dv0, sem_small),
        pltpu.async_copy(d1_tab.at[i_idx], dv1, sem_small),
        pltpu.async_copy(d2_tab.at[i_idx], dv2, sem_small),
        pltpu.async_copy(d3_tab.at[i_idx], dv3, sem_small),
        pltpu.async_copy(d4_tab.at[i_idx], dv4, sem_small),
    ]

    def fix(g, _):
        sl = pl.ds(g * _L, _L)
        v = u_idx[sl]
        hi = v >= nu2
        u_row[sl] = jnp.where(hi, 2 * (v - nu2) + 1, 2 * v)
        w = i_idx[sl]
        hj = w >= ni2
        i_row[sl] = jnp.where(hj, 2 * (w - ni2) + 1, 2 * w)
        return 0

    lax.fori_loop(0, bpw // _L, fix, 0)

    def fire_big(c):
        s = c % 2
        sem = sem_a if s == 0 else sem_b
        sl = pl.ds(c * _CH, _CH)
        return [pltpu.async_copy(ue2.at[u_row.at[sl]], ue_b.at[s], sem),
                pltpu.async_copy(ie2.at[i_row.at[sl]], ie_b.at[s], sem)]

    pend = fire_big(0)

    for c in range(nch):
        for dsc in pend:
            dsc.wait()
        if c + 1 < nch:
            pend = fire_big(c + 1)
        s = c % 2
        uev, iev = ue_b.at[s], ie_b.at[s]
        off = c * _CH

        def group(g, _):
            rows = g * _L + lax.iota(jnp.int32, _L)
            sl16 = pl.ds(off + g * _L, _L)
            zero = jnp.zeros((_L,), jnp.float32)
            accp = [zero, zero]
            accn = [zero, zero]
            for d in range(_D):
                cols = jnp.full((_L,), d, jnp.int32)
                wu = plsc.bitcast(plsc.load_gather(uev, [rows, cols]),
                                  jnp.bfloat16)
                wi = plsc.bitcast(plsc.load_gather(iev, [rows, cols]),
                                  jnp.bfloat16)
                up, un = plsc.unpack(wu, format=plsc.PackFormat.INTERLEAVED)
                ip, in_ = plsc.unpack(wi, format=plsc.PackFormat.INTERLEAVED)
                accp[d % 2] = accp[d % 2] + up * ip
                accn[d % 2] = accn[d % 2] + un * in_
            pos_v[sl16] = accp[0] + accp[1]
            neg_v[sl16] = accn[0] + accn[1]
            return 0

        lax.fori_loop(0, _CH // _L, group, 0)

    for dsc in small:
        dsc.wait()
    osl = pl.ds(base, bpw)
    pltpu.sync_copy(pos_v, pos_out.at[osl])
    pltpu.sync_copy(neg_v, neg_out.at[osl])
    pltpu.sync_copy(rp_v, rpo_out.at[osl])
    pltpu.sync_copy(g_v, gam_out.at[osl])
    pltpu.sync_copy(d_v, dlt_out.at[osl])
    pltpu.sync_copy(up_v, upg_out.at[osl])
    pltpu.sync_copy(un_v, ung_out.at[osl])
    pltpu.sync_copy(ibp_v, ibpg_out.at[osl])
    pltpu.sync_copy(ibn_v, ibng_out.at[osl])
    pltpu.sync_copy(dv0, w0_out.at[osl])
    pltpu.sync_copy(dv1, w1_out.at[osl])
    pltpu.sync_copy(dv2, w2_out.at[osl])
    pltpu.sync_copy(dv3, w3_out.at[osl])
    pltpu.sync_copy(dv4, w4_out.at[osl])


def _tc_body(gbg, gbd, gbp, gbn, pos, neg, upg, ung, ibpg, ibng,
             gam, dlt, rp, q0, q1, q2, q3, q4, out):
    gamma = gbg[0, 0] + gam[...]
    delta = gbd[0, 0] + dlt[...]
    pv = gbp[0, 0] + pos[...] + upg[...] + ibpg[...]
    nv = gbn[0, 0] + neg[...] + ung[...] + ibng[...]
    r = rp[...]
    acc = jnp.zeros_like(r)
    for k, q in enumerate((q0, q1, q2, q3, q4)):
        t = jnp.tanh((k + 1.0) - r)
        val = jnp.where(t > 0.0, pv * t, nv * t)
        dk = q[...]
        num = delta * jnp.exp(gamma * jnp.log(dk))
        den = num + jnp.exp(gamma * jnp.log(1.0 - dk))
        acc = acc + (num / den) * val
    out[...] = acc


def kernel(users, items, distribution, item_price, ref_point, gb_g, ub_g,
           gb_d, ub_d, gb_p, ub_p, ib_p, ue_p, ie_p, gb_n, ub_n, ib_n,
           ue_n, ie_n):
    del item_price
    B = users.shape[0]
    bpw = B // _NW
    NU = ue_p.shape[0]
    NI = ie_p.shape[0]
    nblk = -(-NU // (2 * _BS))
    hb = nblk * _BS
    nu2 = ni2 = hb
    f32 = jnp.float32
    i32 = jnp.int32
    u = users.astype(i32)
    it = items.astype(i32)
    mesh = plsc.VectorSubcoreMesh(core_axis_name="c", subcore_axis_name="s")
    vecs = jax.ShapeDtypeStruct((B,), f32)

    grid = nblk
    half = nblk
    tspec = pl.BlockSpec((_D, _BS), lambda j: (0, j))
    last = (NU - 1) // _BS
    tspec2 = pl.BlockSpec((_D, _BS),
                          lambda j: (0, jnp.minimum(j + half, last)))
    tvec = jax.ShapeDtypeStruct((NI,), f32)
    dspec = pl.BlockSpec((2 * _BS,), lambda j: (j,))
    ue2, ie2, d0, d1, d2, d3, d4 = pl.pallas_call(
        _prep_body,
        grid=(grid,),
        in_specs=[tspec, tspec2, tspec, tspec2, tspec, tspec2, tspec, tspec2,
                  pl.BlockSpec((5, 2 * _BS), lambda j: (0, j))],
        out_specs=[
            pl.BlockSpec((_BS, 2 * _D), lambda j: (j, 0)),
            pl.BlockSpec((_BS, 2 * _D), lambda j: (j, 0)),
            dspec, dspec, dspec, dspec, dspec,
        ],
        out_shape=[
            jax.ShapeDtypeStruct((hb, 2 * _D), i32),
            jax.ShapeDtypeStruct((hb, 2 * _D), i32),
            tvec, tvec, tvec, tvec, tvec,
        ],
    )(ue_p.T, ue_p.T, ue_n.T, ue_n.T, ie_p.T, ie_p.T, ie_n.T, ie_n.T,
      distribution.T)

    sc = pl.kernel(
        functools.partial(_sc_body, nu2=nu2, ni2=ni2),
        out_type=[vecs] * 14,
        mesh=mesh,
        compiler_params=pltpu.CompilerParams(needs_layout_passes=False,
                                             use_tc_tiling_on_sc=False),
        scratch_types=[pltpu.VMEM((bpw,), i32)] * 4
        + [pltpu.VMEM((bpw,), f32)] * 12
        + [pltpu.VMEM((bpw,), f32)] * 2
        + [pltpu.VMEM((2, _CH, _D), i32)] * 2
        + [pltpu.SemaphoreType.DMA] * 3,
    )
    (pos, neg, rpo, gam, dlt, upg, ung, ibpg, ibng,
     w0, w1, w2, w3, w4) = sc(
        u, it, ue2.reshape(2 * hb, _D), ie2.reshape(2 * hb, _D),
        ref_point.reshape(-1), ub_g.reshape(-1), ub_d.reshape(-1),
        ub_p.reshape(-1), ub_n.reshape(-1), ib_p.reshape(-1),
        ib_n.reshape(-1), d0, d1, d2, d3, d4)

    M = B // 128
    r2 = lambda x: x.reshape(M, 128)
    smem = pl.BlockSpec(memory_space=pltpu.SMEM)
    vmem = pl.BlockSpec(memory_space=pltpu.VMEM)
    out2d = pl.pallas_call(
        _tc_body,
        out_shape=jax.ShapeDtypeStruct((M, 128), f32),
        in_specs=[smem] * 4 + [vmem] * 14,
        out_specs=vmem,
    )(gb_g, gb_d, gb_p, gb_n, r2(pos), r2(neg), r2(upg), r2(ung), r2(ibpg),
      r2(ibng), r2(gam), r2(dlt), r2(rpo), r2(w0), r2(w1), r2(w2), r2(w3),
      r2(w4))
    return out2d.reshape(B)


# speedup vs baseline: 1.1214x; 1.1214x over previous
import functools

import jax
import jax.numpy as jnp
from jax import lax
from jax.experimental import pallas as pl
from jax.experimental.pallas import tpu as pltpu
from jax.experimental.pallas import tpu_sc as plsc

_NC = 2
_NS = 16
_NW = _NC * _NS
_L = 16
_D = 64
_CH = 128
_BS = 2048


def _bf16_hi(x):
    u = lax.bitcast_convert_type(x, jnp.uint32)
    r = u + jnp.uint32(0x7FFF) + ((u >> jnp.uint32(16)) & jnp.uint32(1))
    return r & jnp.uint32(0xFFFF0000)


def _prep_body(uepta, uepta2, uenta, uenta2, iepta, iepta2, ienta, ienta2,
               distt, rpt, ugt, udt, upt, unt, ibpt, ibnt,
               ue2, ie2, d0, d1, d2, d3, d4,
               rp1, ug1, ud1, up1, un1, ibp1, ibn1):
    def pack(lo_t, hi_t):
        word = (_bf16_hi(lo_t.T) >> jnp.uint32(16)) | _bf16_hi(hi_t.T)
        return lax.bitcast_convert_type(word, jnp.int32)

    ue2[:, 0:_D] = pack(uepta[...], uenta[...])
    ue2[:, _D:2 * _D] = pack(uepta2[...], uenta2[...])
    ie2[:, 0:_D] = pack(iepta[...], ienta[...])
    ie2[:, _D:2 * _D] = pack(iepta2[...], ienta2[...])
    dd = distt[...]
    d0[...] = dd[0]
    d1[...] = dd[1]
    d2[...] = dd[2]
    d3[...] = dd[3]
    d4[...] = dd[4]
    rp1[...] = rpt[...][0]
    ug1[...] = ugt[...][0]
    ud1[...] = udt[...][0]
    up1[...] = upt[...][0]
    un1[...] = unt[...][0]
    ibp1[...] = ibpt[...][0]
    ibn1[...] = ibnt[...][0]


def _sc_body(users, items, ue2, ie2, rp_tab, ug_tab, ud_tab, up_tab, un_tab,
             ibp_tab, ibn_tab, d0_tab, d1_tab, d2_tab, d3_tab, d4_tab,
             pos_out, neg_out, rpo_out, gam_out, dlt_out, upg_out, ung_out,
             ibpg_out, ibng_out, w0_out, w1_out, w2_out, w3_out, w4_out,
             u_idx, i_idx, u_row, i_row, rp_v, g_v, d_v, up_v, un_v,
             ibp_v, ibn_v, dv0, dv1, dv2, dv3, dv4, pos_v, neg_v,
             ue_b, ie_b, sem_small, sem_a, sem_b, nu2, ni2):
    bpw = pos_v.shape[0]
    nch = bpw // _CH
    wid = lax.axis_index("s") * _NC + lax.axis_index("c")
    base = wid * bpw

    pltpu.sync_copy(users.at[pl.ds(base, bpw)], u_idx)
    pltpu.sync_copy(items.at[pl.ds(base, bpw)], i_idx)

    small = [
        pltpu.async_copy(rp_tab.at[u_idx], rp_v, sem_small),
        pltpu.async_copy(ug_tab.at[u_idx], g_v, sem_small),
        pltpu.async_copy(ud_tab.at[u_idx], d_v, sem_small),
        pltpu.async_copy(up_tab.at[u_idx], up_v, sem_small),
        pltpu.async_copy(un_tab.at[u_idx], un_v, sem_small),
        pltpu.async_copy(ibp_tab.at[i_idx], ibp_v, sem_small),
        pltpu.async_copy(ibn_tab.at[i_idx], ibn_v, sem_small),
        pltpu.async_copy(d0_tab.at[i_idx], dv0, sem_small),
        pltpu.async_copy(d1_tab.at[i_idx], dv1, sem_small),
        pltpu.async_copy(d2_tab.at[i_idx], dv2, sem_small),
        pltpu.async_copy(d3_tab.at[i_idx], dv3, sem_small),
        pltpu.async_copy(d4_tab.at[i_idx], dv4, sem_small),
    ]

    def fix(g, _):
        sl = pl.ds(g * _L, _L)
        v = u_idx[sl]
        hi = v >= nu2
        u_row[sl] = jnp.where(hi, 2 * (v - nu2) + 1, 2 * v)
        w = i_idx[sl]
        hj = w >= ni2
        i_row[sl] = jnp.where(hj, 2 * (w - ni2) + 1, 2 * w)
        return 0

    lax.fori_loop(0, bpw // _L, fix, 0)

    def fire_big(c):
        s = c % 2
        sem = sem_a if s == 0 else sem_b
        sl = pl.ds(c * _CH, _CH)
        return [pltpu.async_copy(ue2.at[u_row.at[sl]], ue_b.at[s], sem),
                pltpu.async_copy(ie2.at[i_row.at[sl]], ie_b.at[s], sem)]

    pend = fire_big(0)

    for c in range(nch):
        for dsc in pend:
            dsc.wait()
        if c + 1 < nch:
            pend = fire_big(c + 1)
        s = c % 2
        uev, iev = ue_b.at[s], ie_b.at[s]
        off = c * _CH

        def group(g, _):
            rows = g * _L + lax.iota(jnp.int32, _L)
            sl16 = pl.ds(off + g * _L, _L)
            zero = jnp.zeros((_L,), jnp.float32)
            accp = [zero, zero]
            accn = [zero, zero]
            for d in range(_D):
                cols = jnp.full((_L,), d, jnp.int32)
                wu = plsc.bitcast(plsc.load_gather(uev, [rows, cols]),
                                  jnp.bfloat16)
                wi = plsc.bitcast(plsc.load_gather(iev, [rows, cols]),
                                  jnp.bfloat16)
                up, un = plsc.unpack(wu, format=plsc.PackFormat.INTERLEAVED)
                ip, in_ = plsc.unpack(wi, format=plsc.PackFormat.INTERLEAVED)
                accp[d % 2] = accp[d % 2] + up * ip
                accn[d % 2] = accn[d % 2] + un * in_
            pos_v[sl16] = accp[0] + accp[1]
            neg_v[sl16] = accn[0] + accn[1]
            return 0

        lax.fori_loop(0, _CH // _L, group, 0)

    for dsc in small:
        dsc.wait()
    osl = pl.ds(base, bpw)
    pltpu.sync_copy(pos_v, pos_out.at[osl])
    pltpu.sync_copy(neg_v, neg_out.at[osl])
    pltpu.sync_copy(rp_v, rpo_out.at[osl])
    pltpu.sync_copy(g_v, gam_out.at[osl])
    pltpu.sync_copy(d_v, dlt_out.at[osl])
    pltpu.sync_copy(up_v, upg_out.at[osl])
    pltpu.sync_copy(un_v, ung_out.at[osl])
    pltpu.sync_copy(ibp_v, ibpg_out.at[osl])
    pltpu.sync_copy(ibn_v, ibng_out.at[osl])
    pltpu.sync_copy(dv0, w0_out.at[osl])
    pltpu.sync_copy(dv1, w1_out.at[osl])
    pltpu.sync_copy(dv2, w2_out.at[osl])
    pltpu.sync_copy(dv3, w3_out.at[osl])
    pltpu.sync_copy(dv4, w4_out.at[osl])


def _tc_body(gbg, gbd, gbp, gbn, pos, neg, upg, ung, ibpg, ibng,
             gam, dlt, rp, q0, q1, q2, q3, q4, out):
    gamma = gbg[0, 0] + gam[...]
    delta = gbd[0, 0] + dlt[...]
    pv = gbp[0, 0] + pos[...] + upg[...] + ibpg[...]
    nv = gbn[0, 0] + neg[...] + ung[...] + ibng[...]
    r = rp[...]
    acc = jnp.zeros_like(r)
    for k, q in enumerate((q0, q1, q2, q3, q4)):
        t = jnp.tanh((k + 1.0) - r)
        val = jnp.where(t > 0.0, pv * t, nv * t)
        dk = q[...]
        num = delta * jnp.exp(gamma * jnp.log(dk))
        den = num + jnp.exp(gamma * jnp.log(1.0 - dk))
        acc = acc + (num / den) * val
    out[...] = acc


def kernel(users, items, distribution, item_price, ref_point, gb_g, ub_g,
           gb_d, ub_d, gb_p, ub_p, ib_p, ue_p, ie_p, gb_n, ub_n, ib_n,
           ue_n, ie_n):
    del item_price
    B = users.shape[0]
    bpw = B // _NW
    NU = ue_p.shape[0]
    NI = ie_p.shape[0]
    nblk = -(-NU // (2 * _BS))
    hb = nblk * _BS
    nu2 = ni2 = hb
    f32 = jnp.float32
    i32 = jnp.int32
    u = users.astype(i32)
    it = items.astype(i32)
    mesh = plsc.VectorSubcoreMesh(core_axis_name="c", subcore_axis_name="s")
    vecs = jax.ShapeDtypeStruct((B,), f32)

    grid = nblk
    half = nblk
    tspec = pl.BlockSpec((_D, _BS), lambda j: (0, j))
    last = (NU - 1) // _BS
    tspec2 = pl.BlockSpec((_D, _BS),
                          lambda j: (0, jnp.minimum(j + half, last)))
    tvec = jax.ShapeDtypeStruct((NI,), f32)
    dspec = pl.BlockSpec((2 * _BS,), lambda j: (j,))
    bspec = pl.BlockSpec((1, 2 * _BS), lambda j: (0, j))
    (ue2, ie2, d0, d1, d2, d3, d4,
     rp1, ug1, ud1, up1, un1, ibp1, ibn1) = pl.pallas_call(
        _prep_body,
        grid=(grid,),
        in_specs=[tspec, tspec2, tspec, tspec2, tspec, tspec2, tspec, tspec2,
                  pl.BlockSpec((5, 2 * _BS), lambda j: (0, j)),
                  bspec, bspec, bspec, bspec, bspec, bspec, bspec],
        out_specs=[
            pl.BlockSpec((_BS, 2 * _D), lambda j: (j, 0)),
            pl.BlockSpec((_BS, 2 * _D), lambda j: (j, 0)),
            dspec, dspec, dspec, dspec, dspec,
            dspec, dspec, dspec, dspec, dspec, dspec, dspec,
        ],
        out_shape=[
            jax.ShapeDtypeStruct((hb, 2 * _D), i32),
            jax.ShapeDtypeStruct((hb, 2 * _D), i32),
            tvec, tvec, tvec, tvec, tvec,
            tvec, tvec, tvec, tvec, tvec, tvec, tvec,
        ],
    )(ue_p.T, ue_p.T, ue_n.T, ue_n.T, ie_p.T, ie_p.T, ie_n.T, ie_n.T,
      distribution.T, ref_point.T, ub_g.T, ub_d.T, ub_p.T, ub_n.T,
      ib_p.T, ib_n.T)

    sc = pl.kernel(
        functools.partial(_sc_body, nu2=nu2, ni2=ni2),
        out_type=[vecs] * 14,
        mesh=mesh,
        compiler_params=pltpu.CompilerParams(needs_layout_passes=False,
                                             use_tc_tiling_on_sc=False),
        scratch_types=[pltpu.VMEM((bpw,), i32)] * 4
        + [pltpu.VMEM((bpw,), f32)] * 12
        + [pltpu.VMEM((bpw,), f32)] * 2
        + [pltpu.VMEM((2, _CH, _D), i32)] * 2
        + [pltpu.SemaphoreType.DMA] * 3,
    )
    (pos, neg, rpo, gam, dlt, upg, ung, ibpg, ibng,
     w0, w1, w2, w3, w4) = sc(
        u, it, ue2.reshape(2 * hb, _D), ie2.reshape(2 * hb, _D),
        rp1, ug1, ud1, up1, un1, ibp1, ibn1, d0, d1, d2, d3, d4)

    M = B // 128
    r2 = lambda x: x.reshape(M, 128)
    smem = pl.BlockSpec(memory_space=pltpu.SMEM)
    vmem = pl.BlockSpec(memory_space=pltpu.VMEM)
    out2d = pl.pallas_call(
        _tc_body,
        out_shape=jax.ShapeDtypeStruct((M, 128), f32),
        in_specs=[smem] * 4 + [vmem] * 14,
        out_specs=vmem,
    )(gb_g, gb_d, gb_p, gb_n, r2(pos), r2(neg), r2(upg), r2(ung), r2(ibpg),
      r2(ibng), r2(gam), r2(dlt), r2(rpo), r2(w0), r2(w1), r2(w2), r2(w3),
      r2(w4))
    return out2d.reshape(B)
